# baseline (speedup 1.0000x reference)
import functools

import jax
import jax.numpy as jnp
from jax import lax
from jax.experimental import pallas as pl
from jax.experimental.pallas import tpu as pltpu
from jax.experimental.pallas import tpu_sc as plsc

F, B, D, K = 8, 1024, 32, 8192
BT = 256
NB = B // BT

NC, NS = 2, 16
NW = NC * NS
BPW = (F * B) // NW
ICH = 128
NCH = BPW // ICH


CW = 512


def _argmin_body(x_ref, w_ref, idx_ref, wt_ref, w2_ref):
    f = pl.program_id(0)
    b = pl.program_id(1)
    x = x_ref[0]
    w = w_ref[0]

    @pl.when(b == 0)
    def _():
        wt_ref[0] = w.T
        w2_ref[...] = jnp.sum(w * w, axis=0, keepdims=True)

    xw2 = lax.dot_general(
        x + x, w, (((1,), (0,)), ((), ())), preferred_element_type=jnp.float32
    )
    x2 = jnp.sum(x * x, axis=1, keepdims=True)

    best = (x2 - xw2[:, :CW]) + w2_ref[0, :CW]
    bj = jnp.zeros(best.shape, jnp.int32)
    for j in range(1, K // CW):
        dj = (x2 - xw2[:, j * CW:(j + 1) * CW]) + w2_ref[0, j * CW:(j + 1) * CW]
        m = dj < best
        best = jnp.where(m, dj, best)
        bj = jnp.where(m, j, bj)

    dmin = jnp.min(best, axis=1, keepdims=True)
    lane = lax.broadcasted_iota(jnp.int32, best.shape, 1)
    candk = jnp.where(best == dmin, bj * CW + lane, 2 * K)
    ki = jnp.min(candk, axis=1, keepdims=True)
    idx_ref[0] = ki + f * K


def _gather_body(table_hbm, idx_hbm, out_hbm, idx_v, rows_v, sem):
    wid = lax.axis_index("s") * NC + lax.axis_index("c")
    pltpu.sync_copy(idx_hbm.at[pl.ds(wid * NCH, NCH)], idx_v)
    copies = [
        pltpu.async_copy(
            table_hbm.at[idx_v.at[j]], rows_v.at[pl.ds(j * ICH, ICH)], sem
        )
        for j in range(NCH)
    ]
    for cp in copies:
        cp.wait()
    pltpu.sync_copy(rows_v, out_hbm.at[pl.ds(wid * BPW, BPW)])


@functools.partial(jax.jit, static_argnames=("interpret",))
def kernel(inputs, W, interpret=False):
    idx, wt = pl.pallas_call(
        _argmin_body,
        grid=(F, NB),
        in_specs=[
            pl.BlockSpec((1, BT, D), lambda f, b: (f, b, 0)),
            pl.BlockSpec((1, D, K), lambda f, b: (f, 0, 0)),
        ],
        out_specs=[
            pl.BlockSpec((1, BT, 1), lambda f, b: (f * NB + b, 0, 0)),
            pl.BlockSpec((1, K, D), lambda f, b: (f, 0, 0)),
        ],
        out_shape=[
            jax.ShapeDtypeStruct((F * NB, BT, 1), jnp.int32),
            jax.ShapeDtypeStruct((F, K, D), jnp.float32),
        ],
        scratch_shapes=[pltpu.VMEM((1, K), jnp.float32)],
        compiler_params=pltpu.CompilerParams(
            dimension_semantics=("parallel", "arbitrary"),
        ),
        interpret=interpret,
    )(inputs, W)

    idx2d = idx.reshape(NW * NCH, ICH)
    wt2d = wt.reshape(F * K, D)
    if interpret:
        q = wt2d[idx2d.reshape(-1)]
        return q.reshape(F, B, D)

    gather = functools.partial(
        pl.kernel,
        mesh=plsc.VectorSubcoreMesh(core_axis_name="c", subcore_axis_name="s"),
        out_type=jax.ShapeDtypeStruct((F * B, D), jnp.float32),
        scratch_types=[
            pltpu.VMEM((NCH, ICH), jnp.int32),
            pltpu.VMEM((BPW, D), jnp.float32),
            pltpu.SemaphoreType.DMA,
        ],
        compiler_params=pltpu.CompilerParams(use_tc_tiling_on_sc=False),
    )(_gather_body)
    q = gather(wt2d, idx2d)
    return q.reshape(F, B, D)


# speedup vs baseline: 1.1161x; 1.1161x over previous
import functools

import jax
import jax.numpy as jnp
from jax import lax
from jax.experimental import pallas as pl
from jax.experimental.pallas import tpu as pltpu
from jax.experimental.pallas import tpu_sc as plsc

F, B, D, K = 8, 1024, 32, 8192
BT = 256
NB = B // BT

NC, NS = 2, 16
NW = NC * NS
BPW = (F * B) // NW
ICH = 128
NCH = BPW // ICH


CW = 256
RG = 64
NG = BT // RG


def _argmin_body(x_ref, w_ref, idx_ref, wt_ref, w2b_ref):
    f = pl.program_id(0)
    b = pl.program_id(1)
    x = x_ref[0]
    w = w_ref[0]

    @pl.when(b == 0)
    def _():
        wt_ref[0] = w.T
        w2 = jnp.sum(w * w, axis=0, keepdims=True)
        w2b_ref[...] = jnp.broadcast_to(w2, (RG, K))

    xw2 = lax.dot_general(
        x + x, w, (((1,), (0,)), ((), ())), preferred_element_type=jnp.float32
    )
    x2 = jnp.sum(x * x, axis=1, keepdims=True)

    lane = lax.broadcasted_iota(jnp.int32, (RG, CW), 1)
    outs = []
    for g in range(NG):
        rows = slice(g * RG, (g + 1) * RG)
        x2b = jnp.broadcast_to(x2[rows], (RG, CW))
        best = (x2b - xw2[rows, :CW]) + w2b_ref[:, :CW]
        bj = jnp.zeros((RG, CW), jnp.int32)
        for j in range(1, K // CW):
            cols = slice(j * CW, (j + 1) * CW)
            dj = (x2b - xw2[rows, cols]) + w2b_ref[:, cols]
            m = dj < best
            best = jnp.where(m, dj, best)
            bj = jnp.where(m, j, bj)
        dmin = jnp.min(best, axis=1, keepdims=True)
        candk = jnp.where(best == dmin, bj * CW + lane, 2 * K)
        outs.append(jnp.min(candk, axis=1, keepdims=True))
    ki = jnp.concatenate(outs, axis=0) + f * K
    idx_ref[0] = jnp.broadcast_to(ki, (BT, 128))


def _gather_body(table_hbm, idx_hbm, out_hbm, idx_v, rows_v, sem):
    wid = lax.axis_index("s") * NC + lax.axis_index("c")
    pltpu.sync_copy(idx_hbm.at[pl.ds(wid * NCH, NCH)], idx_v)
    copies = [
        pltpu.async_copy(
            table_hbm.at[idx_v.at[j]], rows_v.at[pl.ds(j * ICH, ICH)], sem
        )
        for j in range(NCH)
    ]
    for cp in copies:
        cp.wait()
    pltpu.sync_copy(rows_v, out_hbm.at[pl.ds(wid * BPW, BPW)])


@functools.partial(jax.jit, static_argnames=("interpret",))
def kernel(inputs, W, interpret=False):
    idx, wt = pl.pallas_call(
        _argmin_body,
        grid=(F, NB),
        in_specs=[
            pl.BlockSpec((1, BT, D), lambda f, b: (f, b, 0)),
            pl.BlockSpec((1, D, K), lambda f, b: (f, 0, 0)),
        ],
        out_specs=[
            pl.BlockSpec((1, BT, 128), lambda f, b: (f * NB + b, 0, 0)),
            pl.BlockSpec((1, K, D), lambda f, b: (f, 0, 0)),
        ],
        out_shape=[
            jax.ShapeDtypeStruct((F * NB, BT, 128), jnp.int32),
            jax.ShapeDtypeStruct((F, K, D), jnp.float32),
        ],
        scratch_shapes=[pltpu.VMEM((RG, K), jnp.float32)],
        compiler_params=pltpu.CompilerParams(
            dimension_semantics=("parallel", "arbitrary"),
        ),
        interpret=interpret,
    )(inputs, W)

    idx2d = idx[:, :, 0].reshape(NW * NCH, ICH)
    wt2d = wt.reshape(F * K, D)
    if interpret:
        q = wt2d[idx2d.reshape(-1)]
        return q.reshape(F, B, D)

    gather = functools.partial(
        pl.kernel,
        mesh=plsc.VectorSubcoreMesh(core_axis_name="c", subcore_axis_name="s"),
        out_type=jax.ShapeDtypeStruct((F * B, D), jnp.float32),
        scratch_types=[
            pltpu.VMEM((NCH, ICH), jnp.int32),
            pltpu.VMEM((BPW, D), jnp.float32),
            pltpu.SemaphoreType.DMA,
        ],
        compiler_params=pltpu.CompilerParams(use_tc_tiling_on_sc=False),
    )(_gather_body)
    q = gather(wt2d, idx2d)
    return q.reshape(F, B, D)


# speedup vs baseline: 1.1936x; 1.0695x over previous
import functools

import jax
import jax.numpy as jnp
from jax import lax
from jax.experimental import pallas as pl
from jax.experimental.pallas import tpu as pltpu
from jax.experimental.pallas import tpu_sc as plsc

F, B, D, K = 8, 1024, 32, 8192
BT = 256
NB = B // BT

NC, NS = 2, 16
NW = NC * NS
BPW = (F * B) // NW
ICH = 128
NCH = BPW // ICH


CW = 128
RG = 64
NG = BT // RG
KS = K // NB


def _argmin_body(x_ref, w_ref, idx_ref, wt_ref, w2b_ref):
    f = pl.program_id(0)
    b = pl.program_id(1)
    x = x_ref[0]
    w = w_ref[0]

    wt_ref[0] = w_ref[0, :, pl.ds(b * KS, KS)].T

    @pl.when(b == 0)
    def _():
        w2 = jnp.sum(w * w, axis=0, keepdims=True)
        w2b_ref[...] = jnp.broadcast_to(w2, (RG, K))

    xw2 = lax.dot_general(
        x + x, w, (((1,), (0,)), ((), ())), preferred_element_type=jnp.float32
    )
    x2 = jnp.sum(x * x, axis=1, keepdims=True)

    lane = lax.broadcasted_iota(jnp.int32, (RG, CW), 1)
    outs = []
    for g in range(NG):
        rows = slice(g * RG, (g + 1) * RG)
        x2b = jnp.broadcast_to(x2[rows], (RG, CW))
        best = (x2b - xw2[rows, :CW]) + w2b_ref[:, :CW]
        bj = jnp.zeros((RG, CW), jnp.int32)
        for j in range(1, K // CW):
            cols = slice(j * CW, (j + 1) * CW)
            dj = (x2b - xw2[rows, cols]) + w2b_ref[:, cols]
            m = dj < best
            best = jnp.where(m, dj, best)
            bj = jnp.where(m, j, bj)
        dmin = jnp.min(best, axis=1, keepdims=True)
        candk = jnp.where(best == dmin, bj * CW + lane, 2 * K)
        outs.append(jnp.min(candk, axis=1, keepdims=True))
    ki = jnp.concatenate(outs, axis=0) + f * K
    idx_ref[0] = ki.T


def _gather_body(table_hbm, idx_hbm, out_hbm, idx_v, rows_v, sem):
    wid = lax.axis_index("s") * NC + lax.axis_index("c")
    pltpu.sync_copy(idx_hbm.at[pl.ds(wid * NCH, NCH)], idx_v)
    copies = [
        pltpu.async_copy(
            table_hbm.at[idx_v.at[j]], rows_v.at[pl.ds(j * ICH, ICH)], sem
        )
        for j in range(NCH)
    ]
    for cp in copies:
        cp.wait()
    pltpu.sync_copy(rows_v, out_hbm.at[pl.ds(wid * BPW, BPW)])


@functools.partial(jax.jit, static_argnames=("interpret",))
def kernel(inputs, W, interpret=False):
    idx, wt = pl.pallas_call(
        _argmin_body,
        grid=(F, NB),
        in_specs=[
            pl.BlockSpec((1, BT, D), lambda f, b: (f, b, 0)),
            pl.BlockSpec((1, D, K), lambda f, b: (f, 0, 0)),
        ],
        out_specs=[
            pl.BlockSpec((1, 1, BT), lambda f, b: (f * NB + b, 0, 0)),
            pl.BlockSpec((1, KS, D), lambda f, b: (f, b, 0)),
        ],
        out_shape=[
            jax.ShapeDtypeStruct((F * NB, 1, BT), jnp.int32),
            jax.ShapeDtypeStruct((F, K, D), jnp.float32),
        ],
        scratch_shapes=[pltpu.VMEM((RG, K), jnp.float32)],
        compiler_params=pltpu.CompilerParams(
            dimension_semantics=("parallel", "arbitrary"),
        ),
        interpret=interpret,
    )(inputs, W)

    idx2d = idx.reshape(NW * NCH, ICH)
    wt2d = wt.reshape(F * K, D)
    if interpret:
        q = wt2d[idx2d.reshape(-1)]
        return q.reshape(F, B, D)

    gather = functools.partial(
        pl.kernel,
        mesh=plsc.VectorSubcoreMesh(core_axis_name="c", subcore_axis_name="s"),
        out_type=jax.ShapeDtypeStruct((F * B, D), jnp.float32),
        scratch_types=[
            pltpu.VMEM((NCH, ICH), jnp.int32),
            pltpu.VMEM((BPW, D), jnp.float32),
            pltpu.SemaphoreType.DMA,
        ],
        compiler_params=pltpu.CompilerParams(use_tc_tiling_on_sc=False),
    )(_gather_body)
    q = gather(wt2d, idx2d)
    return q.reshape(F, B, D)


# speedup vs baseline: 1.2656x; 1.0603x over previous
import functools

import jax
import jax.numpy as jnp
from jax import lax
from jax.experimental import pallas as pl
from jax.experimental.pallas import tpu as pltpu
from jax.experimental.pallas import tpu_sc as plsc

F, B, D, K = 8, 1024, 32, 8192
BT = 512
NB = B // BT

NC, NS = 2, 16
NW = NC * NS
BPW = (F * B) // NW
ICH = 128
NCH = BPW // ICH


CW = 128
RG = 64
NG = BT // RG
KS = K // NB


def _argmin_body(x_ref, w_ref, idx_ref, wt_ref, w2b_ref):
    f = pl.program_id(0)
    b = pl.program_id(1)
    x = x_ref[0]
    w = w_ref[0]

    wt_ref[0] = w_ref[0, :, pl.ds(b * KS, KS)].T

    @pl.when(b == 0)
    def _():
        w2 = jnp.sum(w * w, axis=0, keepdims=True)
        w2b_ref[...] = jnp.broadcast_to(w2, (RG, K))

    xw2 = lax.dot_general(
        x + x, w, (((1,), (0,)), ((), ())), preferred_element_type=jnp.float32
    )
    x2 = jnp.sum(x * x, axis=1, keepdims=True)

    lane = lax.broadcasted_iota(jnp.int32, (RG, CW), 1)
    outs = []
    for g in range(NG):
        rows = slice(g * RG, (g + 1) * RG)
        x2b = jnp.broadcast_to(x2[rows], (RG, CW))
        best = (x2b - xw2[rows, :CW]) + w2b_ref[:, :CW]
        bj = jnp.zeros((RG, CW), jnp.int32)
        for j in range(1, K // CW):
            cols = slice(j * CW, (j + 1) * CW)
            dj = (x2b - xw2[rows, cols]) + w2b_ref[:, cols]
            m = dj < best
            best = jnp.where(m, dj, best)
            bj = jnp.where(m, j, bj)
        dmin = jnp.min(best, axis=1, keepdims=True)
        candk = jnp.where(best == dmin, bj * CW + lane, 2 * K)
        outs.append(jnp.min(candk, axis=1, keepdims=True))
    ki = jnp.concatenate(outs, axis=0) + f * K
    idx_ref[0] = ki.T


def _gather_body(table_hbm, idx_hbm, out_hbm, idx_v, rows_v, sem):
    wid = lax.axis_index("s") * NC + lax.axis_index("c")
    pltpu.sync_copy(idx_hbm.at[pl.ds(wid * NCH, NCH)], idx_v)
    copies = [
        pltpu.async_copy(
            table_hbm.at[idx_v.at[j]], rows_v.at[pl.ds(j * ICH, ICH)], sem
        )
        for j in range(NCH)
    ]
    for cp in copies:
        cp.wait()
    pltpu.sync_copy(rows_v, out_hbm.at[pl.ds(wid * BPW, BPW)])


@functools.partial(jax.jit, static_argnames=("interpret",))
def kernel(inputs, W, interpret=False):
    idx, wt = pl.pallas_call(
        _argmin_body,
        grid=(F, NB),
        in_specs=[
            pl.BlockSpec((1, BT, D), lambda f, b: (f, b, 0)),
            pl.BlockSpec((1, D, K), lambda f, b: (f, 0, 0)),
        ],
        out_specs=[
            pl.BlockSpec((1, 1, BT), lambda f, b: (f * NB + b, 0, 0)),
            pl.BlockSpec((1, KS, D), lambda f, b: (f, b, 0)),
        ],
        out_shape=[
            jax.ShapeDtypeStruct((F * NB, 1, BT), jnp.int32),
            jax.ShapeDtypeStruct((F, K, D), jnp.float32),
        ],
        scratch_shapes=[pltpu.VMEM((RG, K), jnp.float32)],
        compiler_params=pltpu.CompilerParams(
            dimension_semantics=("parallel", "arbitrary"),
        ),
        interpret=interpret,
    )(inputs, W)

    idx2d = idx.reshape(NW * NCH, ICH)
    wt2d = wt.reshape(F * K, D)
    if interpret:
        q = wt2d[idx2d.reshape(-1)]
        return q.reshape(F, B, D)

    gather = functools.partial(
        pl.kernel,
        mesh=plsc.VectorSubcoreMesh(core_axis_name="c", subcore_axis_name="s"),
        out_type=jax.ShapeDtypeStruct((F * B, D), jnp.float32),
        scratch_types=[
            pltpu.VMEM((NCH, ICH), jnp.int32),
            pltpu.VMEM((BPW, D), jnp.float32),
            pltpu.SemaphoreType.DMA,
        ],
        compiler_params=pltpu.CompilerParams(use_tc_tiling_on_sc=False),
    )(_gather_body)
    q = gather(wt2d, idx2d)
    return q.reshape(F, B, D)


# speedup vs baseline: 1.3664x; 1.0797x over previous
import functools

import jax
import jax.numpy as jnp
from jax import lax
from jax.experimental import pallas as pl
from jax.experimental.pallas import tpu as pltpu
from jax.experimental.pallas import tpu_sc as plsc

F, B, D, K = 8, 1024, 32, 8192
BT = 1024
NB = B // BT

NC, NS = 2, 16
NW = NC * NS
BPW = (F * B) // NW
ICH = 128
NCH = BPW // ICH


CW = 128
RG = 64
NG = BT // RG
KS = K // NB


def _argmin_body(x_ref, w_ref, idx_ref, wt_ref, w2b_ref):
    f = pl.program_id(0)
    b = pl.program_id(1)
    x = x_ref[0]
    w = w_ref[0]

    wt_ref[0] = w_ref[0, :, pl.ds(b * KS, KS)].T

    @pl.when(b == 0)
    def _():
        w2 = jnp.sum(w * w, axis=0, keepdims=True)
        w2b_ref[...] = jnp.broadcast_to(w2, (RG, K))

    xw2 = lax.dot_general(
        x + x, w, (((1,), (0,)), ((), ())), preferred_element_type=jnp.float32
    )
    x2 = jnp.sum(x * x, axis=1, keepdims=True)

    lane = lax.broadcasted_iota(jnp.int32, (RG, CW), 1)
    outs = []
    for g in range(NG):
        rows = slice(g * RG, (g + 1) * RG)
        x2b = jnp.broadcast_to(x2[rows], (RG, CW))
        best = (x2b - xw2[rows, :CW]) + w2b_ref[:, :CW]
        bj = jnp.zeros((RG, CW), jnp.int32)
        for j in range(1, K // CW):
            cols = slice(j * CW, (j + 1) * CW)
            dj = (x2b - xw2[rows, cols]) + w2b_ref[:, cols]
            m = dj < best
            best = jnp.where(m, dj, best)
            bj = jnp.where(m, j, bj)
        dmin = jnp.min(best, axis=1, keepdims=True)
        candk = jnp.where(best == dmin, bj * CW + lane, 2 * K)
        outs.append(jnp.min(candk, axis=1, keepdims=True))
    ki = jnp.concatenate(outs, axis=0) + f * K
    idx_ref[0] = ki.T


def _gather_body(table_hbm, idx_hbm, out_hbm, idx_v, rows_v, sem):
    wid = lax.axis_index("s") * NC + lax.axis_index("c")
    pltpu.sync_copy(idx_hbm.at[pl.ds(wid * NCH, NCH)], idx_v)
    copies = [
        pltpu.async_copy(
            table_hbm.at[idx_v.at[j]], rows_v.at[pl.ds(j * ICH, ICH)], sem
        )
        for j in range(NCH)
    ]
    for cp in copies:
        cp.wait()
    pltpu.sync_copy(rows_v, out_hbm.at[pl.ds(wid * BPW, BPW)])


@functools.partial(jax.jit, static_argnames=("interpret",))
def kernel(inputs, W, interpret=False):
    idx, wt = pl.pallas_call(
        _argmin_body,
        grid=(F, NB),
        in_specs=[
            pl.BlockSpec((1, BT, D), lambda f, b: (f, b, 0)),
            pl.BlockSpec((1, D, K), lambda f, b: (f, 0, 0)),
        ],
        out_specs=[
            pl.BlockSpec((1, 1, BT), lambda f, b: (f * NB + b, 0, 0)),
            pl.BlockSpec((1, KS, D), lambda f, b: (f, b, 0)),
        ],
        out_shape=[
            jax.ShapeDtypeStruct((F * NB, 1, BT), jnp.int32),
            jax.ShapeDtypeStruct((F, K, D), jnp.float32),
        ],
        scratch_shapes=[pltpu.VMEM((RG, K), jnp.float32)],
        compiler_params=pltpu.CompilerParams(
            dimension_semantics=("parallel", "arbitrary"),
        ),
        interpret=interpret,
    )(inputs, W)

    idx2d = idx.reshape(NW * NCH, ICH)
    wt2d = wt.reshape(F * K, D)
    if interpret:
        q = wt2d[idx2d.reshape(-1)]
        return q.reshape(F, B, D)

    gather = functools.partial(
        pl.kernel,
        mesh=plsc.VectorSubcoreMesh(core_axis_name="c", subcore_axis_name="s"),
        out_type=jax.ShapeDtypeStruct((F * B, D), jnp.float32),
        scratch_types=[
            pltpu.VMEM((NCH, ICH), jnp.int32),
            pltpu.VMEM((BPW, D), jnp.float32),
            pltpu.SemaphoreType.DMA,
        ],
        compiler_params=pltpu.CompilerParams(use_tc_tiling_on_sc=False),
    )(_gather_body)
    q = gather(wt2d, idx2d)
    return q.reshape(F, B, D)


# speedup vs baseline: 1.4182x; 1.0379x over previous
import functools

import jax
import jax.numpy as jnp
from jax import lax
from jax.experimental import pallas as pl
from jax.experimental.pallas import tpu as pltpu
from jax.experimental.pallas import tpu_sc as plsc

F, B, D, K = 8, 1024, 32, 8192
BT = 1024
NB = B // BT

NC, NS = 2, 16
NW = NC * NS
BPW = (F * B) // NW
ICH = 128
NCH = BPW // ICH


CW = 128
RG = 64
NG = BT // RG
KS = K // NB


def _argmin_body(x_ref, w_ref, idx_ref, wt_ref, w2b_ref):
    f = pl.program_id(0)
    b = pl.program_id(1)
    x = x_ref[0]
    w = w_ref[0]

    wt_ref[0] = w_ref[0, :, pl.ds(b * KS, KS)].T

    @pl.when(b == 0)
    def _():
        w2 = jnp.sum(w * w, axis=0, keepdims=True)
        w2b_ref[...] = jnp.broadcast_to(w2, (RG, K))

    xd = x + x
    x2 = jnp.sum(x * x, axis=1, keepdims=True)

    lane = lax.broadcasted_iota(jnp.int32, (RG, CW), 1)
    outs = []
    for g in range(NG):
        rows = slice(g * RG, (g + 1) * RG)
        xw2 = lax.dot_general(
            xd[rows], w, (((1,), (0,)), ((), ())),
            preferred_element_type=jnp.float32,
        )
        x2b = jnp.broadcast_to(x2[rows], (RG, CW))
        best = (x2b - xw2[:, :CW]) + w2b_ref[:, :CW]
        bj = jnp.zeros((RG, CW), jnp.int32)
        for j in range(1, K // CW):
            cols = slice(j * CW, (j + 1) * CW)
            dj = (x2b - xw2[:, cols]) + w2b_ref[:, cols]
            m = dj < best
            best = jnp.where(m, dj, best)
            bj = jnp.where(m, j, bj)
        dmin = jnp.min(best, axis=1, keepdims=True)
        candk = jnp.where(best == dmin, bj * CW + lane, 2 * K)
        outs.append(jnp.min(candk, axis=1, keepdims=True))
    ki = jnp.concatenate(outs, axis=0) + f * K
    idx_ref[0] = ki.T


def _gather_body(table_hbm, idx_hbm, out_hbm, idx_v, rows_v, sem):
    wid = lax.axis_index("s") * NC + lax.axis_index("c")
    pltpu.sync_copy(idx_hbm.at[pl.ds(wid * NCH, NCH)], idx_v)
    copies = [
        pltpu.async_copy(
            table_hbm.at[idx_v.at[j]], rows_v.at[pl.ds(j * ICH, ICH)], sem
        )
        for j in range(NCH)
    ]
    for cp in copies:
        cp.wait()
    pltpu.sync_copy(rows_v, out_hbm.at[pl.ds(wid * BPW, BPW)])


@functools.partial(jax.jit, static_argnames=("interpret",))
def kernel(inputs, W, interpret=False):
    idx, wt = pl.pallas_call(
        _argmin_body,
        grid=(F, NB),
        in_specs=[
            pl.BlockSpec((1, BT, D), lambda f, b: (f, b, 0)),
            pl.BlockSpec((1, D, K), lambda f, b: (f, 0, 0)),
        ],
        out_specs=[
            pl.BlockSpec((1, 1, BT), lambda f, b: (f * NB + b, 0, 0)),
            pl.BlockSpec((1, KS, D), lambda f, b: (f, b, 0)),
        ],
        out_shape=[
            jax.ShapeDtypeStruct((F * NB, 1, BT), jnp.int32),
            jax.ShapeDtypeStruct((F, K, D), jnp.float32),
        ],
        scratch_shapes=[pltpu.VMEM((RG, K), jnp.float32)],
        compiler_params=pltpu.CompilerParams(
            dimension_semantics=("parallel", "arbitrary"),
        ),
        interpret=interpret,
    )(inputs, W)

    idx2d = idx.reshape(NW * NCH, ICH)
    wt2d = wt.reshape(F * K, D)
    if interpret:
        q = wt2d[idx2d.reshape(-1)]
        return q.reshape(F, B, D)

    gather = functools.partial(
        pl.kernel,
        mesh=plsc.VectorSubcoreMesh(core_axis_name="c", subcore_axis_name="s"),
        out_type=jax.ShapeDtypeStruct((F * B, D), jnp.float32),
        scratch_types=[
            pltpu.VMEM((NCH, ICH), jnp.int32),
            pltpu.VMEM((BPW, D), jnp.float32),
            pltpu.SemaphoreType.DMA,
        ],
        compiler_params=pltpu.CompilerParams(use_tc_tiling_on_sc=False),
    )(_gather_body)
    q = gather(wt2d, idx2d)
    return q.reshape(F, B, D)


# speedup vs baseline: 1.4710x; 1.0373x over previous
import functools

import jax
import jax.numpy as jnp
from jax import lax
from jax.experimental import pallas as pl
from jax.experimental.pallas import tpu as pltpu
from jax.experimental.pallas import tpu_sc as plsc

F, B, D, K = 8, 1024, 32, 8192
BT = 1024
NB = B // BT

NC, NS = 2, 16
NW = NC * NS
BPW = (F * B) // NW
ICH = 128
NCH = BPW // ICH


CW = 128
RG = 128
NG = BT // RG
KS = K // NB


def _argmin_body(x_ref, w_ref, idx_ref, wt_ref, w2b_ref):
    f = pl.program_id(0)
    b = pl.program_id(1)
    x = x_ref[0]
    w = w_ref[0]

    wt_ref[0] = w_ref[0, :, pl.ds(b * KS, KS)].T

    @pl.when(b == 0)
    def _():
        w2 = jnp.sum(w * w, axis=0, keepdims=True)
        w2b_ref[...] = jnp.broadcast_to(w2, (RG, K))

    xd = x + x
    x2 = jnp.sum(x * x, axis=1, keepdims=True)

    lane = lax.broadcasted_iota(jnp.int32, (RG, CW), 1)
    outs = []
    for g in range(NG):
        rows = slice(g * RG, (g + 1) * RG)
        xw2 = lax.dot_general(
            xd[rows], w, (((1,), (0,)), ((), ())),
            preferred_element_type=jnp.float32,
        )
        x2b = jnp.broadcast_to(x2[rows], (RG, CW))
        best = (x2b - xw2[:, :CW]) + w2b_ref[:, :CW]
        bj = jnp.zeros((RG, CW), jnp.int32)
        for j in range(1, K // CW):
            cols = slice(j * CW, (j + 1) * CW)
            dj = (x2b - xw2[:, cols]) + w2b_ref[:, cols]
            m = dj < best
            best = jnp.where(m, dj, best)
            bj = jnp.where(m, j, bj)
        dmin = jnp.min(best, axis=1, keepdims=True)
        candk = jnp.where(best == dmin, bj * CW + lane, 2 * K)
        outs.append(jnp.min(candk, axis=1, keepdims=True))
    ki = jnp.concatenate(outs, axis=0) + f * K
    idx_ref[0] = ki.T


def _gather_body(table_hbm, idx_hbm, out_hbm, idx_v, rows_v, sem):
    wid = lax.axis_index("s") * NC + lax.axis_index("c")
    pltpu.sync_copy(idx_hbm.at[pl.ds(wid * NCH, NCH)], idx_v)
    copies = [
        pltpu.async_copy(
            table_hbm.at[idx_v.at[j]], rows_v.at[pl.ds(j * ICH, ICH)], sem
        )
        for j in range(NCH)
    ]
    for cp in copies:
        cp.wait()
    pltpu.sync_copy(rows_v, out_hbm.at[pl.ds(wid * BPW, BPW)])


@functools.partial(jax.jit, static_argnames=("interpret",))
def kernel(inputs, W, interpret=False):
    idx, wt = pl.pallas_call(
        _argmin_body,
        grid=(F, NB),
        in_specs=[
            pl.BlockSpec((1, BT, D), lambda f, b: (f, b, 0)),
            pl.BlockSpec((1, D, K), lambda f, b: (f, 0, 0)),
        ],
        out_specs=[
            pl.BlockSpec((1, 1, BT), lambda f, b: (f * NB + b, 0, 0)),
            pl.BlockSpec((1, KS, D), lambda f, b: (f, b, 0)),
        ],
        out_shape=[
            jax.ShapeDtypeStruct((F * NB, 1, BT), jnp.int32),
            jax.ShapeDtypeStruct((F, K, D), jnp.float32),
        ],
        scratch_shapes=[pltpu.VMEM((RG, K), jnp.float32)],
        compiler_params=pltpu.CompilerParams(
            dimension_semantics=("parallel", "arbitrary"),
        ),
        interpret=interpret,
    )(inputs, W)

    idx2d = idx.reshape(NW * NCH, ICH)
    wt2d = wt.reshape(F * K, D)
    if interpret:
        q = wt2d[idx2d.reshape(-1)]
        return q.reshape(F, B, D)

    gather = functools.partial(
        pl.kernel,
        mesh=plsc.VectorSubcoreMesh(core_axis_name="c", subcore_axis_name="s"),
        out_type=jax.ShapeDtypeStruct((F * B, D), jnp.float32),
        scratch_types=[
            pltpu.VMEM((NCH, ICH), jnp.int32),
            pltpu.VMEM((BPW, D), jnp.float32),
            pltpu.SemaphoreType.DMA,
        ],
        compiler_params=pltpu.CompilerParams(use_tc_tiling_on_sc=False),
    )(_gather_body)
    q = gather(wt2d, idx2d)
    return q.reshape(F, B, D)
